# baseline (speedup 1.0000x reference)
import functools

import jax
import jax.numpy as jnp
from jax import lax
from jax.experimental import pallas as pl
from jax.experimental.pallas import tpu as pltpu
from jax.experimental.pallas import tpu_sc as plsc


def _gather_kernel(B, D, b_per_w, chunk, n_chunks, NC):
    mesh = plsc.VectorSubcoreMesh(core_axis_name="c", subcore_axis_name="s")

    @functools.partial(
        pl.kernel,
        mesh=mesh,
        out_type=jax.ShapeDtypeStruct((B, D), jnp.float32),
        scratch_types=[
            pltpu.VMEM((b_per_w,), jnp.int32),
            pltpu.VMEM((chunk, D), jnp.float32),
            pltpu.SemaphoreType.DMA,
        ],
        compiler_params=pltpu.CompilerParams(use_tc_tiling_on_sc=False),
    )
    def k(idx_hbm, table_hbm, out_hbm, idx_v, rows_v, sem):
        wid = lax.axis_index("s") * NC + lax.axis_index("c")
        base = wid * b_per_w
        pltpu.sync_copy(idx_hbm.at[pl.ds(base, b_per_w)], idx_v)
        for g in range(n_chunks):
            pltpu.async_copy(
                table_hbm.at[idx_v.at[pl.ds(g * chunk, chunk)]], rows_v, sem
            ).wait()
            pltpu.sync_copy(rows_v, out_hbm.at[pl.ds(base + g * chunk, chunk)])

    return k


def kernel(input, weight):
    B0, B1 = input.shape
    V, D = weight.shape
    B = B0 * B1

    info = plsc.get_sparse_core_info()
    NC, NS = info.num_cores, info.num_subcores
    NW = NC * NS
    b_per_w = B // NW
    chunk = 800
    n_chunks = b_per_w // chunk

    idx_flat = input.reshape(B).astype(jnp.int32)
    out = _gather_kernel(B, D, b_per_w, chunk, n_chunks, NC)(idx_flat, weight)
    return out.reshape(B0, B1, D)


# speedup vs baseline: 1.0029x; 1.0029x over previous
import functools

import jax
import jax.numpy as jnp
from jax import lax
from jax.experimental import pallas as pl
from jax.experimental.pallas import tpu as pltpu
from jax.experimental.pallas import tpu_sc as plsc


def _gather_kernel(B, D, b_per_w, chunk, n_chunks, NC):
    mesh = plsc.VectorSubcoreMesh(core_axis_name="c", subcore_axis_name="s")

    @functools.partial(
        pl.kernel,
        mesh=mesh,
        out_type=jax.ShapeDtypeStruct((B, D), jnp.float32),
        scratch_types=[
            pltpu.VMEM((b_per_w,), jnp.int32),
            pltpu.VMEM((chunk, D), jnp.float32),
            pltpu.VMEM((chunk, D), jnp.float32),
            pltpu.SemaphoreType.DMA,
            pltpu.SemaphoreType.DMA,
        ],
        compiler_params=pltpu.CompilerParams(use_tc_tiling_on_sc=False),
    )
    def k(idx_hbm, table_hbm, out_hbm, idx_v, rows0, rows1, sem_g, sem_w):
        wid = lax.axis_index("s") * NC + lax.axis_index("c")
        base = wid * b_per_w
        pltpu.sync_copy(idx_hbm.at[pl.ds(base, b_per_w)], idx_v)
        bufs = (rows0, rows1)

        def gather(g):
            return pltpu.async_copy(
                table_hbm.at[idx_v.at[pl.ds(g * chunk, chunk)]],
                bufs[g % 2],
                sem_g,
            )

        def write(g):
            return pltpu.async_copy(
                bufs[g % 2],
                out_hbm.at[pl.ds(base + g * chunk, chunk)],
                sem_w,
            )

        gathers = [gather(g) for g in range(0, 1)]
        gathers[0].wait()
        writes = []
        for g in range(n_chunks):
            if g + 1 < n_chunks:
                if g >= 1:
                    writes[g - 1].wait()
                nxt = gather(g + 1)
            writes.append(write(g))
            if g + 1 < n_chunks:
                nxt.wait()
        writes[n_chunks - 2].wait()
        writes[n_chunks - 1].wait()

    return k


def kernel(input, weight):
    B0, B1 = input.shape
    V, D = weight.shape
    B = B0 * B1

    info = plsc.get_sparse_core_info()
    NC, NS = info.num_cores, info.num_subcores
    NW = NC * NS
    b_per_w = B // NW
    chunk = 800
    n_chunks = b_per_w // chunk

    idx_flat = input.reshape(B).astype(jnp.int32)
    out = _gather_kernel(B, D, b_per_w, chunk, n_chunks, NC)(idx_flat, weight)
    return out.reshape(B0, B1, D)
